# Initial kernel scaffold; baseline (speedup 1.0000x reference)
import jax
import jax.numpy as jnp
from jax.experimental import pallas as pl


def kernel(x, edge_index, W1, att_l1, att_r1, b1, W2, att_l2, att_r2, b2, W3, att_l3, att_r3, b3):
    raise NotImplementedError("write your pallas kernel here")


# speedup vs baseline: 8.5794x; 8.5794x over previous
import jax
import jax.numpy as jnp
from jax import lax
from jax.experimental import pallas as pl
from jax.experimental.pallas import tpu as pltpu
from jax.experimental.pallas import tpu_sc as plsc

N = 10000
D = 128
E = 320000
EP = E + N
NW = 32
CK = 128
EA = 10368
EPP = EA * NW
NP = 10240
NPS = NP // 16
CB = 256
EB = EPP // 16
NEG = -3.0e38

_mesh = plsc.VectorSubcoreMesh(core_axis_name="c", subcore_axis_name="s")


def _tc_first_body(h_ref, w_ref, attl_ref, attr_ref, xp_ref, al_ref, ar_ref):
    xp = jnp.dot(h_ref[...], w_ref[...], preferred_element_type=jnp.float32)
    xp_ref[...] = xp
    al_ref[...] = jnp.sum(xp * attl_ref[...], axis=1, keepdims=True)
    ar_ref[...] = jnp.sum(xp * attr_ref[...], axis=1, keepdims=True)


def _tc_next_body(p_ref, b_ref, w_ref, attl_ref, attr_ref,
                  xp_ref, al_ref, ar_ref):
    h = jnp.maximum(p_ref[0] + p_ref[1] + b_ref[...], 0.0)
    xp = jnp.dot(h, w_ref[...], preferred_element_type=jnp.float32)
    xp_ref[...] = xp
    al_ref[...] = jnp.sum(xp * attl_ref[...], axis=1, keepdims=True)
    ar_ref[...] = jnp.sum(xp * attr_ref[...], axis=1, keepdims=True)


_BR = 1024


def _tc_first(h, W, attl, attr):
    do = W.shape[1]
    return pl.pallas_call(
        _tc_first_body,
        grid=(NP // _BR,),
        in_specs=[pl.BlockSpec((_BR, D), lambda i: (i, 0)),
                  pl.BlockSpec((D, do), lambda i: (0, 0)),
                  pl.BlockSpec((1, do), lambda i: (0, 0)),
                  pl.BlockSpec((1, do), lambda i: (0, 0))],
        out_specs=[pl.BlockSpec((_BR, do), lambda i: (i, 0)),
                   pl.BlockSpec((_BR, 1), lambda i: (i, 0)),
                   pl.BlockSpec((_BR, 1), lambda i: (i, 0))],
        out_shape=[jax.ShapeDtypeStruct((NP, do), jnp.float32),
                   jax.ShapeDtypeStruct((NP, 1), jnp.float32),
                   jax.ShapeDtypeStruct((NP, 1), jnp.float32)],
    )(h, W, attl, attr)


def _tc_next(p, b, W, attl, attr):
    do = W.shape[1]
    return pl.pallas_call(
        _tc_next_body,
        grid=(NP // _BR,),
        in_specs=[pl.BlockSpec((2, _BR, D), lambda i: (0, i, 0)),
                  pl.BlockSpec((1, D), lambda i: (0, 0)),
                  pl.BlockSpec((D, do), lambda i: (0, 0)),
                  pl.BlockSpec((1, do), lambda i: (0, 0)),
                  pl.BlockSpec((1, do), lambda i: (0, 0))],
        out_specs=[pl.BlockSpec((_BR, do), lambda i: (i, 0)),
                   pl.BlockSpec((_BR, 1), lambda i: (i, 0)),
                   pl.BlockSpec((_BR, 1), lambda i: (i, 0))],
        out_shape=[jax.ShapeDtypeStruct((NP, do), jnp.float32),
                   jax.ShapeDtypeStruct((NP, 1), jnp.float32),
                   jax.ShapeDtypeStruct((NP, 1), jnp.float32)],
    )(p, b, W, attl, attr)


def _tc_final_body(q_ref, b_ref, o_ref):
    o_ref[...] = q_ref[0:1, :] + q_ref[1:2, :] + b_ref[0, 0]


def _tc_final(q, b):
    BL = 2048
    return pl.pallas_call(
        _tc_final_body,
        grid=(NP // BL,),
        in_specs=[pl.BlockSpec((2, BL), lambda i: (0, i)),
                  pl.BlockSpec((1, 1), lambda i: (0, 0))],
        out_specs=pl.BlockSpec((1, BL), lambda i: (0, i)),
        out_shape=jax.ShapeDtypeStruct((1, NP), jnp.float32),
    )(q, b)


_LANE = None


def _fill(ref, val, n16):
    c = jnp.full((16,), val, ref.dtype)

    def body(i, _):
        ref[pl.ds(i * 16, 16)] = c
        return 0

    lax.fori_loop(0, n16, body, 0)


_DN = lax.GatherDimensionNumbers(offset_dims=(), collapsed_slice_dims=(0,),
                                 start_index_map=(0,))


def _perm(v, idx):
    return lax.gather(v, idx[:, None], _DN, (1,),
                      mode=lax.GatherScatterMode.PROMISE_IN_BOUNDS)


def _hsum_all(v, lane):
    for sh in (8, 4, 2, 1):
        v = v + _perm(v, lane ^ sh)
    return v


def _hmax_all(v, lane):
    for sh in (8, 4, 2, 1):
        v = jnp.maximum(v, _perm(v, lane ^ sh))
    return v


def _edge_alpha(lvec, alv, arv):
    z = alv + arv
    sg = 1.0 / (1.0 + jnp.exp(-lvec))
    av = z * sg
    return jnp.where(av >= 0.0, av, 0.2 * av)


def _core_max_combine(mx_b, t16_b, stage_sh, sid, cid, maxp_o):
    pltpu.sync_copy(mx_b, stage_sh.at[sid])
    plsc.subcore_barrier()

    @pl.when(sid == 0)
    def _():
        pltpu.sync_copy(stage_sh.at[0], mx_b)

        def jb(j, _):
            pltpu.sync_copy(stage_sh.at[j], t16_b)
            mx_b[pl.ds(0, 16)] = jnp.maximum(mx_b[pl.ds(0, 16)],
                                             t16_b[pl.ds(0, 16)])
            return 0

        lax.fori_loop(1, 16, jb, 0)
        pltpu.sync_copy(mx_b, maxp_o.at[cid])


def _load_mg(maxp, m16_b, t16_b, lane):
    pltpu.sync_copy(maxp.at[0], m16_b)
    pltpu.sync_copy(maxp.at[1], t16_b)
    m = jnp.maximum(m16_b[pl.ds(0, 16)], t16_b[pl.ds(0, 16)])
    return _hmax_all(m, lane)


def _sc_alpha_body(xp, al, ar, src, dst, alpha_o, maxp_o,
                   sidx_v, didx_v, xj_v, xi_v, als_b, ars_b, abuf,
                   mx_b, t16_b, stage_sh, sem_j, sem_i, sem_a, sem_r):
    cid = lax.axis_index("c")
    sid = lax.axis_index("s")
    wid = sid * 2 + cid
    lane = lax.iota(jnp.int32, 16)
    _fill(mx_b, NEG, 1)

    def chunk(c, _):
        base = wid * EA + c * CK
        pltpu.sync_copy(src.at[pl.ds(base, CK)], sidx_v)
        pltpu.sync_copy(dst.at[pl.ds(base, CK)], didx_v)
        cpj = pltpu.async_copy(xp.at[sidx_v], xj_v, sem_j)
        cpi = pltpu.async_copy(xp.at[didx_v], xi_v, sem_i)
        cpa = pltpu.async_copy(al.at[sidx_v], als_b, sem_a)
        cpr = pltpu.async_copy(ar.at[didx_v], ars_b, sem_r)
        cpj.wait()
        cpi.wait()
        cpa.wait()
        cpr.wait()

        def grp(g, _):
            lvec = jnp.zeros((16,), jnp.float32)
            for e in range(16):
                r = g * 16 + e
                acc = xi_v[r, pl.ds(0, 16)] * xj_v[r, pl.ds(0, 16)]
                for s in range(1, 8):
                    acc = acc + (xi_v[r, pl.ds(s * 16, 16)]
                                 * xj_v[r, pl.ds(s * 16, 16)])
                lvec = jnp.where(lane == e, _hsum_all(acc, lane), lvec)
            alv = als_b[pl.ds(g * 16, 16)]
            arv = ars_b[pl.ds(g * 16, 16)]
            av = _edge_alpha(lvec, alv, arv)
            abuf[pl.ds(g * 16, 16)] = av
            mx_b[pl.ds(0, 16)] = jnp.maximum(mx_b[pl.ds(0, 16)], av)
            return 0

        lax.fori_loop(0, CK // 16, grp, 0)
        pltpu.sync_copy(abuf, alpha_o.at[pl.ds(base, CK)])
        return 0

    lax.fori_loop(0, EA // CK, chunk, 0)
    _core_max_combine(mx_b, t16_b, stage_sh, sid, cid, maxp_o)


def _sc_alpha(xp, al, ar, src, dst):
    f = pl.kernel(
        _sc_alpha_body,
        out_type=[jax.ShapeDtypeStruct((EPP,), jnp.float32),
                  jax.ShapeDtypeStruct((2, 16), jnp.float32)],
        mesh=_mesh,
        scratch_types=[
            pltpu.VMEM((CK,), jnp.int32),
            pltpu.VMEM((CK,), jnp.int32),
            pltpu.VMEM((CK, D), jnp.float32),
            pltpu.VMEM((CK, D), jnp.float32),
            pltpu.VMEM((CK,), jnp.float32),
            pltpu.VMEM((CK,), jnp.float32),
            pltpu.VMEM((CK,), jnp.float32),
            pltpu.VMEM((16,), jnp.float32),
            pltpu.VMEM((16,), jnp.float32),
            pltpu.VMEM_SHARED((16, 16), jnp.float32),
            pltpu.SemaphoreType.DMA,
            pltpu.SemaphoreType.DMA,
            pltpu.SemaphoreType.DMA,
            pltpu.SemaphoreType.DMA,
        ],
    )
    return f(xp, al, ar, src, dst)


def _sc_out_body(xp, src, dst, alpha, maxp, outp,
                 sidx_v, didx_v, didx_b, alpha_b, exb,
                 rows_v, ach_b, den_b, wb, m16_b, t16_b, cz_b,
                 den_sh, acc_sh, sem, semd):
    cid = lax.axis_index("c")
    sid = lax.axis_index("s")
    wid = sid * 2 + cid
    lane = lax.iota(jnp.int32, 16)
    mgv = _load_mg(maxp, m16_b, t16_b, lane)

    _fill(cz_b, 0.0, NPS // 16)
    pltpu.sync_copy(cz_b, den_sh.at[pl.ds(sid * NPS, NPS)])
    z16 = jnp.zeros((16,), jnp.float32)

    def zr(i, _):
        for s in range(8):
            rows_v[i, pl.ds(s * 16, 16)] = z16
        return 0

    lax.fori_loop(0, CK, zr, 0)

    def zc(k, _):
        pltpu.sync_copy(rows_v, acc_sh.at[pl.ds(sid * NPS + k * CK, CK), :])
        return 0

    lax.fori_loop(0, NPS // CK, zc, 0)
    plsc.subcore_barrier()

    def bchunk(c, _):
        base = sid * EB + c * CB
        pltpu.sync_copy(dst.at[pl.ds(base, CB)], didx_b)
        pltpu.sync_copy(alpha.at[pl.ds(base, CB)], alpha_b)

        def bv(v, _):
            av = alpha_b[pl.ds(v * 16, 16)]
            exb[pl.ds(v * 16, 16)] = jnp.exp(av - mgv)
            return 0

        lax.fori_loop(0, CB // 16, bv, 0)
        pltpu.sync_copy(exb, den_sh.at[didx_b], add=True)
        return 0

    lax.fori_loop(0, EB // CB, bchunk, 0)
    plsc.subcore_barrier()

    def cchunk(c, _):
        base = wid * EA + c * CK
        pltpu.sync_copy(src.at[pl.ds(base, CK)], sidx_v)
        pltpu.sync_copy(dst.at[pl.ds(base, CK)], didx_v)
        pltpu.sync_copy(alpha.at[pl.ds(base, CK)], ach_b)
        cpr = pltpu.async_copy(xp.at[sidx_v], rows_v, sem)
        cpd = pltpu.async_copy(den_sh.at[didx_v], den_b, semd)
        cpr.wait()
        cpd.wait()

        def cg(g, _):
            av = ach_b[pl.ds(g * 16, 16)]
            dv = den_b[pl.ds(g * 16, 16)]
            wb[pl.ds(g * 16, 16)] = jnp.exp(av - mgv) / (dv + 1e-16)
            return 0

        lax.fori_loop(0, CK // 16, cg, 0)

        def sg(g, _):
            wv = wb[pl.ds(g * 16, 16)]
            for e in range(16):
                r = g * 16 + e
                w = wv[e]
                for s in range(8):
                    rows_v[r, pl.ds(s * 16, 16)] = (
                        rows_v[r, pl.ds(s * 16, 16)] * w)
            return 0

        lax.fori_loop(0, CK // 16, sg, 0)
        pltpu.sync_copy(rows_v, acc_sh.at[didx_v], add=True)
        return 0

    lax.fori_loop(0, EA // CK, cchunk, 0)
    plsc.subcore_barrier()
    pltpu.sync_copy(acc_sh.at[pl.ds(sid * NPS, NPS), :],
                    outp.at[cid, pl.ds(sid * NPS, NPS), :])


def _sc_out(xp, src, dst, alpha, maxp):
    f = pl.kernel(
        _sc_out_body,
        out_type=jax.ShapeDtypeStruct((2, NP, D), jnp.float32),
        mesh=_mesh,
        scratch_types=[
            pltpu.VMEM((CK,), jnp.int32),
            pltpu.VMEM((CK,), jnp.int32),
            pltpu.VMEM((CB,), jnp.int32),
            pltpu.VMEM((CB,), jnp.float32),
            pltpu.VMEM((CB,), jnp.float32),
            pltpu.VMEM((CK, D), jnp.float32),
            pltpu.VMEM((CK,), jnp.float32),
            pltpu.VMEM((CK,), jnp.float32),
            pltpu.VMEM((CK,), jnp.float32),
            pltpu.VMEM((16,), jnp.float32),
            pltpu.VMEM((16,), jnp.float32),
            pltpu.VMEM((NPS,), jnp.float32),
            pltpu.VMEM_SHARED((NP,), jnp.float32),
            pltpu.VMEM_SHARED((NP, D), jnp.float32),
            pltpu.SemaphoreType.DMA,
            pltpu.SemaphoreType.DMA,
        ],
    )
    return f(xp, src, dst, alpha, maxp)


def _sc_alpha3_body(x3, al3, ar3, src, dst, alpha_o, maxp_o,
                    sidx_v, didx_v, x3s_b, x3d_b, als_b, ars_b, abuf,
                    mx_b, t16_b, stage_sh, sem1, sem2, sem3, sem4):
    cid = lax.axis_index("c")
    sid = lax.axis_index("s")
    wid = sid * 2 + cid
    _fill(mx_b, NEG, 1)

    def chunk(c, _):
        base = wid * EA + c * CK
        pltpu.sync_copy(src.at[pl.ds(base, CK)], sidx_v)
        pltpu.sync_copy(dst.at[pl.ds(base, CK)], didx_v)
        cp1 = pltpu.async_copy(x3.at[sidx_v], x3s_b, sem1)
        cp2 = pltpu.async_copy(x3.at[didx_v], x3d_b, sem2)
        cp3 = pltpu.async_copy(al3.at[sidx_v], als_b, sem3)
        cp4 = pltpu.async_copy(ar3.at[didx_v], ars_b, sem4)
        cp1.wait()
        cp2.wait()
        cp3.wait()
        cp4.wait()

        def grp(g, _):
            sl = pl.ds(g * 16, 16)
            lvec = x3s_b[sl] * x3d_b[sl]
            av = _edge_alpha(lvec, als_b[sl], ars_b[sl])
            abuf[sl] = av
            mx_b[pl.ds(0, 16)] = jnp.maximum(mx_b[pl.ds(0, 16)], av)
            return 0

        lax.fori_loop(0, CK // 16, grp, 0)
        pltpu.sync_copy(abuf, alpha_o.at[pl.ds(base, CK)])
        return 0

    lax.fori_loop(0, EA // CK, chunk, 0)
    _core_max_combine(mx_b, t16_b, stage_sh, sid, cid, maxp_o)


def _sc_alpha3(x3, al3, ar3, src, dst):
    f = pl.kernel(
        _sc_alpha3_body,
        out_type=[jax.ShapeDtypeStruct((EPP,), jnp.float32),
                  jax.ShapeDtypeStruct((2, 16), jnp.float32)],
        mesh=_mesh,
        scratch_types=[
            pltpu.VMEM((CK,), jnp.int32),
            pltpu.VMEM((CK,), jnp.int32),
            pltpu.VMEM((CK,), jnp.float32),
            pltpu.VMEM((CK,), jnp.float32),
            pltpu.VMEM((CK,), jnp.float32),
            pltpu.VMEM((CK,), jnp.float32),
            pltpu.VMEM((CK,), jnp.float32),
            pltpu.VMEM((16,), jnp.float32),
            pltpu.VMEM((16,), jnp.float32),
            pltpu.VMEM_SHARED((16, 16), jnp.float32),
            pltpu.SemaphoreType.DMA,
            pltpu.SemaphoreType.DMA,
            pltpu.SemaphoreType.DMA,
            pltpu.SemaphoreType.DMA,
        ],
    )
    return f(x3, al3, ar3, src, dst)


def _sc_out3_body(x3, src, dst, alpha, maxp, outp,
                  sidx_v, didx_v, didx_b, alpha_b, exb,
                  x3s_b, ach_b, den_b, cv_b, m16_b, t16_b, cz_b,
                  den_sh, acc3_sh, sem, semd):
    cid = lax.axis_index("c")
    sid = lax.axis_index("s")
    wid = sid * 2 + cid
    lane = lax.iota(jnp.int32, 16)
    mgv = _load_mg(maxp, m16_b, t16_b, lane)

    _fill(cz_b, 0.0, NPS // 16)
    pltpu.sync_copy(cz_b, den_sh.at[pl.ds(sid * NPS, NPS)])
    pltpu.sync_copy(cz_b, acc3_sh.at[pl.ds(sid * NPS, NPS)])
    plsc.subcore_barrier()

    def bchunk(c, _):
        base = sid * EB + c * CB
        pltpu.sync_copy(dst.at[pl.ds(base, CB)], didx_b)
        pltpu.sync_copy(alpha.at[pl.ds(base, CB)], alpha_b)

        def bv(v, _):
            av = alpha_b[pl.ds(v * 16, 16)]
            exb[pl.ds(v * 16, 16)] = jnp.exp(av - mgv)
            return 0

        lax.fori_loop(0, CB // 16, bv, 0)
        pltpu.sync_copy(exb, den_sh.at[didx_b], add=True)
        return 0

    lax.fori_loop(0, EB // CB, bchunk, 0)
    plsc.subcore_barrier()

    def cchunk(c, _):
        base = wid * EA + c * CK
        pltpu.sync_copy(src.at[pl.ds(base, CK)], sidx_v)
        pltpu.sync_copy(dst.at[pl.ds(base, CK)], didx_v)
        pltpu.sync_copy(alpha.at[pl.ds(base, CK)], ach_b)
        cp1 = pltpu.async_copy(x3.at[sidx_v], x3s_b, sem)
        cp2 = pltpu.async_copy(den_sh.at[didx_v], den_b, semd)
        cp1.wait()
        cp2.wait()

        def cg(g, _):
            sl = pl.ds(g * 16, 16)
            w = jnp.exp(ach_b[sl] - mgv) / (den_b[sl] + 1e-16)
            cv_b[sl] = w * x3s_b[sl]
            return 0

        lax.fori_loop(0, CK // 16, cg, 0)
        pltpu.sync_copy(cv_b, acc3_sh.at[didx_v], add=True)
        return 0

    lax.fori_loop(0, EA // CK, cchunk, 0)
    plsc.subcore_barrier()
    pltpu.sync_copy(acc3_sh.at[pl.ds(sid * NPS, NPS)],
                    outp.at[cid, pl.ds(sid * NPS, NPS)])


def _sc_out3(x3, src, dst, alpha, maxp):
    f = pl.kernel(
        _sc_out3_body,
        out_type=jax.ShapeDtypeStruct((2, NP), jnp.float32),
        mesh=_mesh,
        scratch_types=[
            pltpu.VMEM((CK,), jnp.int32),
            pltpu.VMEM((CK,), jnp.int32),
            pltpu.VMEM((CB,), jnp.int32),
            pltpu.VMEM((CB,), jnp.float32),
            pltpu.VMEM((CB,), jnp.float32),
            pltpu.VMEM((CK,), jnp.float32),
            pltpu.VMEM((CK,), jnp.float32),
            pltpu.VMEM((CK,), jnp.float32),
            pltpu.VMEM((CK,), jnp.float32),
            pltpu.VMEM((16,), jnp.float32),
            pltpu.VMEM((16,), jnp.float32),
            pltpu.VMEM((NPS,), jnp.float32),
            pltpu.VMEM_SHARED((NP,), jnp.float32),
            pltpu.VMEM_SHARED((NP,), jnp.float32),
            pltpu.SemaphoreType.DMA,
            pltpu.SemaphoreType.DMA,
        ],
    )
    return f(x3, src, dst, alpha, maxp)


def kernel(x, edge_index, W1, att_l1, att_r1, b1,
           W2, att_l2, att_r2, b2, W3, att_l3, att_r3, b3):
    loop = jnp.arange(N, dtype=jnp.int32)
    padi = jnp.full((EPP - EP,), N, jnp.int32)
    src = jnp.concatenate([edge_index[0], loop, padi])
    dst = jnp.concatenate([edge_index[1], loop, padi])
    xpad = jnp.pad(x, ((0, NP - N), (0, 0)))

    xp1, al1, ar1 = _tc_first(xpad, W1, att_l1.reshape(1, -1),
                              att_r1.reshape(1, -1))
    alpha1, maxp1 = _sc_alpha(xp1, al1.reshape(NP), ar1.reshape(NP), src, dst)
    p1 = _sc_out(xp1, src, dst, alpha1, maxp1)

    xp2, al2, ar2 = _tc_next(p1, b1.reshape(1, -1), W2,
                             att_l2.reshape(1, -1), att_r2.reshape(1, -1))
    alpha2, maxp2 = _sc_alpha(xp2, al2.reshape(NP), ar2.reshape(NP), src, dst)
    p2 = _sc_out(xp2, src, dst, alpha2, maxp2)

    xp3, al3, ar3 = _tc_next(p2, b2.reshape(1, -1), W3,
                             att_l3.reshape(1, 1), att_r3.reshape(1, 1))
    x3v = xp3.reshape(NP)
    alpha3, maxp3 = _sc_alpha3(x3v, al3.reshape(NP), ar3.reshape(NP),
                               src, dst)
    q = _sc_out3(x3v, src, dst, alpha3, maxp3)

    o = _tc_final(q, b3.reshape(1, 1))
    return o.reshape(NP, 1)[:N]
